# baseline (speedup 1.0000x reference)
import jax
import jax.numpy as jnp
from jax.experimental import pallas as pl
from jax.experimental.pallas import tpu as pltpu

_B = 16
_BLK = 1024


def _total(n_ref):
    t = n_ref[0]
    for k in range(1, _B):
        t = t + n_ref[k]
    return t


def _outer(n_ref, x_hbm, w1_ref, b1_ref, w2_ref, b2_ref, out_ref,
           acc_ref, starts_ref, ends_ref, cnt_ref, w1b_ref, h_buf_ref):
    acc_ref[...] = jnp.zeros_like(acc_ref)
    cnt_ref[0] = 0
    w1b_ref[...] = w1_ref[...].astype(jnp.bfloat16)

    row = jax.lax.broadcasted_iota(jnp.int32, (_B, 1), 0)
    starts = jnp.zeros((_B, 1), jnp.int32)
    ends = jnp.zeros((_B, 1), jnp.int32)
    e = n_ref[0]
    ends = jnp.where(row == 0, e, ends)
    for k in range(1, _B):
        s = e
        e = e + n_ref[k]
        starts = jnp.where(row == k, s, starts)
        ends = jnp.where(row == k, e, ends)
    starts_ref[...] = starts
    ends_ref[...] = ends

    total = _total(n_ref)
    nsteps = jnp.maximum(pl.cdiv(total, _BLK), 1)

    def _aggregate(c):
        t_row = (c * _BLK
                 + jax.lax.broadcasted_iota(jnp.int32, (_B, _BLK), 1))
        onehot = ((t_row >= starts_ref[...])
                  & (t_row < ends_ref[...])).astype(jnp.bfloat16)
        acc_ref[...] += jax.lax.dot_general(
            onehot, h_buf_ref[c % 2], (((1,), (0,)), ((), ())),
            preferred_element_type=jnp.float32,
        )

    def inner(x_ref):
        c = cnt_ref[0]
        @pl.when(c > 0)
        def _():
            _aggregate(c - 1)

        h_buf_ref[c % 2] = jnp.maximum(
            jnp.dot(
                x_ref[...].astype(jnp.bfloat16),
                w1b_ref[...],
                preferred_element_type=jnp.float32,
            )
            + b1_ref[...],
            0.0,
        ).astype(jnp.bfloat16)
        cnt_ref[0] = c + 1

    pltpu.emit_pipeline(
        inner,
        grid=(nsteps,),
        in_specs=[pl.BlockSpec((_BLK, x_hbm.shape[1]), lambda i: (i, 0),
                               pipeline_mode=pl.Buffered(buffer_count=3))],
    )(x_hbm)

    _aggregate(nsteps - 1)
    out_ref[...] = (
        jnp.dot(acc_ref[...], w2_ref[...], preferred_element_type=jnp.float32)
        + b2_ref[...]
    )


def kernel(x, n_instances, W1, b1, W2, b2):
    tok, d = x.shape
    h = W1.shape[1]
    out_dim = W2.shape[1]

    grid_spec = pltpu.PrefetchScalarGridSpec(
        num_scalar_prefetch=1,
        grid=(1,),
        in_specs=[
            pl.BlockSpec(memory_space=pl.ANY),
            pl.BlockSpec((d, h), lambda i, n: (0, 0)),
            pl.BlockSpec((1, h), lambda i, n: (0, 0)),
            pl.BlockSpec((h, out_dim), lambda i, n: (0, 0)),
            pl.BlockSpec((1, out_dim), lambda i, n: (0, 0)),
        ],
        out_specs=pl.BlockSpec((_B, out_dim), lambda i, n: (0, 0)),
        scratch_shapes=[
            pltpu.VMEM((_B, h), jnp.float32),
            pltpu.VMEM((_B, 1), jnp.int32),
            pltpu.VMEM((_B, 1), jnp.int32),
            pltpu.SMEM((1,), jnp.int32),
            pltpu.VMEM((d, h), jnp.bfloat16),
            pltpu.VMEM((2, _BLK, h), jnp.bfloat16),
        ],
    )

    return pl.pallas_call(
        _outer,
        grid_spec=grid_spec,
        out_shape=jax.ShapeDtypeStruct((_B, out_dim), jnp.float32),
        compiler_params=pltpu.CompilerParams(
            dimension_semantics=("arbitrary",)),
    )(n_instances, x, W1, b1.reshape(1, -1), W2, b2.reshape(1, -1))


# speedup vs baseline: 1.1197x; 1.1197x over previous
import jax
import jax.numpy as jnp
from jax.experimental import pallas as pl
from jax.experimental.pallas import tpu as pltpu

_B = 16
_BLK = 1536


def _total(n_ref):
    t = n_ref[0]
    for k in range(1, _B):
        t = t + n_ref[k]
    return t


def _outer(n_ref, x_hbm, w1_ref, b1_ref, w2_ref, b2_ref, out_ref,
           acc_ref, starts_ref, ends_ref, cnt_ref, w1b_ref):
    acc_ref[...] = jnp.zeros_like(acc_ref)
    cnt_ref[0] = 0
    w1b_ref[...] = w1_ref[...].astype(jnp.bfloat16)

    row = jax.lax.broadcasted_iota(jnp.int32, (_B, 1), 0)
    starts = jnp.zeros((_B, 1), jnp.int32)
    ends = jnp.zeros((_B, 1), jnp.int32)
    e = n_ref[0]
    ends = jnp.where(row == 0, e, ends)
    for k in range(1, _B):
        s = e
        e = e + n_ref[k]
        starts = jnp.where(row == k, s, starts)
        ends = jnp.where(row == k, e, ends)
    starts_ref[...] = starts
    ends_ref[...] = ends

    total = _total(n_ref)
    nsteps = jnp.maximum(pl.cdiv(total, _BLK), 1)

    def inner(x_ref):
        c = cnt_ref[0]
        h = jnp.maximum(
            jnp.dot(
                x_ref[...].astype(jnp.bfloat16),
                w1b_ref[...],
                preferred_element_type=jnp.float32,
            )
            + b1_ref[...],
            0.0,
        ).astype(jnp.bfloat16)
        t_row = (c * _BLK
                 + jax.lax.broadcasted_iota(jnp.int32, (_B, _BLK), 1))
        onehot = ((t_row >= starts_ref[...])
                  & (t_row < ends_ref[...])).astype(jnp.bfloat16)
        acc_ref[...] += jax.lax.dot_general(
            onehot, h, (((1,), (0,)), ((), ())),
            preferred_element_type=jnp.float32,
        )
        cnt_ref[0] = c + 1

    pltpu.emit_pipeline(
        inner,
        grid=(nsteps,),
        in_specs=[pl.BlockSpec((_BLK, x_hbm.shape[1]), lambda i: (i, 0),
                               pipeline_mode=pl.Buffered(buffer_count=3))],
    )(x_hbm)

    out_ref[...] = (
        jnp.dot(acc_ref[...], w2_ref[...], preferred_element_type=jnp.float32)
        + b2_ref[...]
    )


def kernel(x, n_instances, W1, b1, W2, b2):
    tok, d = x.shape
    h = W1.shape[1]
    out_dim = W2.shape[1]

    grid_spec = pltpu.PrefetchScalarGridSpec(
        num_scalar_prefetch=1,
        grid=(1,),
        in_specs=[
            pl.BlockSpec(memory_space=pl.ANY),
            pl.BlockSpec((d, h), lambda i, n: (0, 0)),
            pl.BlockSpec((1, h), lambda i, n: (0, 0)),
            pl.BlockSpec((h, out_dim), lambda i, n: (0, 0)),
            pl.BlockSpec((1, out_dim), lambda i, n: (0, 0)),
        ],
        out_specs=pl.BlockSpec((_B, out_dim), lambda i, n: (0, 0)),
        scratch_shapes=[
            pltpu.VMEM((_B, h), jnp.float32),
            pltpu.VMEM((_B, 1), jnp.int32),
            pltpu.VMEM((_B, 1), jnp.int32),
            pltpu.SMEM((1,), jnp.int32),
            pltpu.VMEM((d, h), jnp.bfloat16),
        ],
    )

    return pl.pallas_call(
        _outer,
        grid_spec=grid_spec,
        out_shape=jax.ShapeDtypeStruct((_B, out_dim), jnp.float32),
        compiler_params=pltpu.CompilerParams(
            dimension_semantics=("arbitrary",)),
    )(n_instances, x, W1, b1.reshape(1, -1), W2, b2.reshape(1, -1))


# speedup vs baseline: 1.1269x; 1.0064x over previous
import jax
import jax.numpy as jnp
from jax.experimental import pallas as pl
from jax.experimental.pallas import tpu as pltpu

_B = 16
_BLK = 1536


def _total(n_ref):
    t = n_ref[0]
    for k in range(1, _B):
        t = t + n_ref[k]
    return t


def _outer(n_ref, x_hbm, w1_ref, b1_ref, w2_ref, b2_ref, out_ref,
           acc_ref, starts_ref, ends_ref, cnt_ref, w1b_ref):
    acc_ref[...] = jnp.zeros_like(acc_ref)
    cnt_ref[0] = 0
    w1b_ref[...] = w1_ref[...].astype(jnp.bfloat16)

    row = jax.lax.broadcasted_iota(jnp.int32, (_B, 1), 0)
    starts = jnp.zeros((_B, 1), jnp.int32)
    ends = jnp.zeros((_B, 1), jnp.int32)
    e = n_ref[0]
    ends = jnp.where(row == 0, e, ends)
    for k in range(1, _B):
        s = e
        e = e + n_ref[k]
        starts = jnp.where(row == k, s, starts)
        ends = jnp.where(row == k, e, ends)
    starts_ref[...] = starts
    ends_ref[...] = ends

    total = _total(n_ref)
    nsteps = jnp.maximum(pl.cdiv(total, _BLK), 1)

    def inner(x_ref):
        c = cnt_ref[0]
        h = jnp.maximum(
            jnp.dot(
                x_ref[...].astype(jnp.bfloat16),
                w1b_ref[...],
                preferred_element_type=jnp.float32,
            )
            + b1_ref[...],
            0.0,
        ).astype(jnp.bfloat16)
        t_row = (c * _BLK
                 + jax.lax.broadcasted_iota(jnp.int32, (_B, _BLK), 1))
        onehot = ((t_row >= starts_ref[...])
                  & (t_row < ends_ref[...])).astype(jnp.bfloat16)
        acc_ref[...] += jax.lax.dot_general(
            onehot, h, (((1,), (0,)), ((), ())),
            preferred_element_type=jnp.float32,
        )
        cnt_ref[0] = c + 1

    pltpu.emit_pipeline(
        inner,
        grid=(nsteps,),
        in_specs=[pl.BlockSpec((_BLK, x_hbm.shape[1]), lambda i: (i, 0),
                               pipeline_mode=pl.Buffered(buffer_count=4))],
    )(x_hbm)

    out_ref[...] = (
        jnp.dot(acc_ref[...], w2_ref[...], preferred_element_type=jnp.float32)
        + b2_ref[...]
    )


def kernel(x, n_instances, W1, b1, W2, b2):
    tok, d = x.shape
    h = W1.shape[1]
    out_dim = W2.shape[1]

    grid_spec = pltpu.PrefetchScalarGridSpec(
        num_scalar_prefetch=1,
        grid=(1,),
        in_specs=[
            pl.BlockSpec(memory_space=pl.ANY),
            pl.BlockSpec((d, h), lambda i, n: (0, 0)),
            pl.BlockSpec((1, h), lambda i, n: (0, 0)),
            pl.BlockSpec((h, out_dim), lambda i, n: (0, 0)),
            pl.BlockSpec((1, out_dim), lambda i, n: (0, 0)),
        ],
        out_specs=pl.BlockSpec((_B, out_dim), lambda i, n: (0, 0)),
        scratch_shapes=[
            pltpu.VMEM((_B, h), jnp.float32),
            pltpu.VMEM((_B, 1), jnp.int32),
            pltpu.VMEM((_B, 1), jnp.int32),
            pltpu.SMEM((1,), jnp.int32),
            pltpu.VMEM((d, h), jnp.bfloat16),
        ],
    )

    return pl.pallas_call(
        _outer,
        grid_spec=grid_spec,
        out_shape=jax.ShapeDtypeStruct((_B, out_dim), jnp.float32),
        compiler_params=pltpu.CompilerParams(
            dimension_semantics=("arbitrary",)),
    )(n_instances, x, W1, b1.reshape(1, -1), W2, b2.reshape(1, -1))
